# baseline (speedup 1.0000x reference)
import functools

import jax
import jax.numpy as jnp
from jax import lax
from jax.experimental import pallas as pl
from jax.experimental.pallas import tpu as pltpu
from jax.experimental.pallas import tpu_sc as plsc


_NW = 32
_CHUNK = 125
_NCHUNK = 8


def _sc_gather(ids3, reps, C, S, D):
    ncls = -(-C // _NW)

    mesh = plsc.VectorSubcoreMesh(core_axis_name="c", subcore_axis_name="s")

    @functools.partial(
        pl.kernel,
        mesh=mesh,
        out_type=jax.ShapeDtypeStruct((C, S, D), jnp.float32),
        scratch_types=[
            pltpu.VMEM((_NCHUNK, _CHUNK), jnp.int32),
            pltpu.VMEM((S, D), jnp.float32),
            pltpu.SemaphoreType.DMA,
        ],
    )
    def k(ids_hbm, reps_hbm, out_hbm, idx_v, rows_v, sem):
        wid = lax.axis_index("s") * 2 + lax.axis_index("c")

        def cls_body(t, carry):
            c = wid + t * _NW

            @pl.when(c < C)
            def _():
                pltpu.sync_copy(ids_hbm.at[c], idx_v)
                copies = [
                    pltpu.async_copy(
                        reps_hbm.at[idx_v.at[j]],
                        rows_v.at[pl.ds(j * _CHUNK, _CHUNK)],
                        sem,
                    )
                    for j in range(_NCHUNK)
                ]
                for cp in copies:
                    cp.wait()
                pltpu.sync_copy(rows_v, out_hbm.at[c])

            return carry

        lax.fori_loop(0, ncls, cls_body, 0)

    return k(ids3, reps)


def _eye_f32(n):
    r = lax.broadcasted_iota(jnp.int32, (n, n), 0)
    c = lax.broadcasted_iota(jnp.int32, (n, n), 1)
    return (r == c).astype(jnp.float32)


def _tc_body(S, K, g_ref, ids_ref, out_ref):
    G = g_ref[0]
    csum = jnp.sum(G, axis=0, keepdims=True)
    center = csum / jnp.float32(S)

    sim_row = lax.dot_general(
        center, G, (((1,), (1,)), ((), ())))

    eye_s = _eye_f32(S)
    sim_col = lax.dot_general(
        eye_s, sim_row, (((1,), (1,)), ((), ())),
        precision=lax.Precision.HIGHEST)

    ids_row = ids_ref[0].astype(jnp.float32)
    ids_col = lax.dot_general(
        eye_s, ids_row, (((1,), (1,)), ((), ())),
        precision=lax.Precision.HIGHEST)

    i_idx = lax.broadcasted_iota(jnp.int32, (S, S), 0)
    j_idx = lax.broadcasted_iota(jnp.int32, (S, S), 1)
    lt = sim_row < sim_col
    eq = (sim_row == sim_col) & (j_idx < i_idx)
    rank = jnp.sum((lt | eq).astype(jnp.int32), axis=1, keepdims=True)

    r_iota = lax.broadcasted_iota(jnp.int32, (S, K), 1)
    sel = rank == r_iota
    picked = jnp.where(sel, jnp.broadcast_to(ids_col, (S, K)), 0.0)
    out_ref[0] = jnp.sum(picked, axis=0, keepdims=True).astype(jnp.int32)


def _tc_compute(G3, ids3r, C, S, K):
    body = functools.partial(_tc_body, S, K)
    return pl.pallas_call(
        body,
        grid=(C,),
        in_specs=[
            pl.BlockSpec((1, S, G3.shape[2]), lambda c: (c, 0, 0)),
            pl.BlockSpec((1, 1, S), lambda c: (c, 0, 0)),
        ],
        out_specs=pl.BlockSpec((1, 1, K), lambda c: (c, 0, 0)),
        out_shape=jax.ShapeDtypeStruct((C, 1, K), jnp.int32),
    )(G3, ids3r)


def kernel(ids_per_cls_train, budget, feats, reps, d):
    ids = ids_per_cls_train.astype(jnp.int32)
    C, S = ids.shape
    D = reps.shape[1]
    K = min(100, S)
    KPAD = 128

    ids3 = ids.reshape(C, _NCHUNK, _CHUNK)
    G3 = _sc_gather(ids3, reps, C, S, D)

    ids3r = ids.reshape(C, 1, S)
    out = _tc_compute(G3, ids3r, C, S, KPAD)

    return out[:, 0, :K].reshape(-1).astype(ids_per_cls_train.dtype)
